# baseline (speedup 1.0000x reference)
import functools

import jax
import jax.numpy as jnp
from jax import lax
from jax.experimental import pallas as pl
from jax.experimental.pallas import tpu as pltpu
from jax.experimental.pallas import tpu_sc as plsc

N = 10000
D = 128
E = 320000
NC, NS = 2, 16
NW = NC * NS
CH = 128
C = -(-E // (NW * CH))
C += C % 2
EW = C * CH
E_PAD = NW * EW
NP = 10240
SROWS = NP // 16
TOTC = NW * C
CA = C
CB = TOTC // NS - CA

_i32 = jnp.int32
_f32 = jnp.float32


def _leaky(v):
    return jnp.where(v >= 0, v, 0.01 * v)


def _norm_rows(v):
    n = jnp.sqrt(jnp.sum(v * v, axis=-1, keepdims=True))
    return v / jnp.maximum(n, 1e-12)


def _tc_pre_body(x_ref, win_ref, bin_ref, wl_ref, wr_ref, nl_ref, nr_ref,
                 xl_ref):
    h = _leaky(jnp.dot(x_ref[...], win_ref[...],
                       preferred_element_type=_f32) + bin_ref[...])
    xl = jnp.dot(h, wl_ref[...], preferred_element_type=_f32)
    xr = jnp.dot(h, wr_ref[...], preferred_element_type=_f32)
    xl_ref[...] = xl
    nl_ref[...] = _norm_rows(xl)
    nr_ref[...] = _norm_rows(xr)


def _tc_mid_body(op_ref, b_ref, wc_ref, wl_ref, wr_ref, nl_ref, nr_ref,
                 xl_ref):
    o = op_ref[0] + op_ref[1] + b_ref[...]
    h = _leaky(jnp.dot(o, wc_ref[...], preferred_element_type=_f32))
    xl = jnp.dot(h, wl_ref[...], preferred_element_type=_f32)
    xr = jnp.dot(h, wr_ref[...], preferred_element_type=_f32)
    xl_ref[...] = xl
    nl_ref[...] = _norm_rows(xl)
    nr_ref[...] = _norm_rows(xr)


def _tc_fin_body(op_ref, b_ref, wc_ref, wm1_ref, bm1_ref, wm2_ref, bm2_ref,
                 z_ref):
    o = op_ref[0] + op_ref[1] + b_ref[...]
    h = _leaky(jnp.dot(o, wc_ref[...], preferred_element_type=_f32))
    m = jnp.maximum(jnp.dot(h, wm1_ref[...], preferred_element_type=_f32)
                    + bm1_ref[...], 0.0)
    z_ref[...] = (jnp.dot(m, wm2_ref[...], preferred_element_type=_f32)
                  + bm2_ref[...])


_GRID = 16
_RB = NP // _GRID

_row_spec = pl.BlockSpec((_RB, D), lambda i: (i, 0))
_w_spec = pl.BlockSpec((D, D), lambda i: (0, 0))
_b_spec = pl.BlockSpec((1, D), lambda i: (0, 0))
_op_spec = pl.BlockSpec((2, _RB, D), lambda i: (0, i, 0))

_tc_pre = pl.pallas_call(
    _tc_pre_body,
    grid=(_GRID,),
    in_specs=[_row_spec, _w_spec, _b_spec, _w_spec, _w_spec],
    out_specs=[_row_spec, _row_spec, _row_spec],
    out_shape=[jax.ShapeDtypeStruct((NP, D), _f32)] * 3,
)

_oph_spec = pl.BlockSpec((2, _RB, D), lambda i: (0, i, 0))

_tc_mid = pl.pallas_call(
    _tc_mid_body,
    grid=(_GRID,),
    in_specs=[_oph_spec, _b_spec, _w_spec, _w_spec, _w_spec],
    out_specs=[_row_spec, _row_spec, _row_spec],
    out_shape=[jax.ShapeDtypeStruct((NP, D), _f32)] * 3,
)

_tc_fin = pl.pallas_call(
    _tc_fin_body,
    grid=(_GRID,),
    in_specs=[_oph_spec, _b_spec, _w_spec, _w_spec, _b_spec, _w_spec,
              _b_spec],
    out_specs=_row_spec,
    out_shape=jax.ShapeDtypeStruct((NP, D), _f32),
)


def _tc_div_body(xl_ref, ss_ref, xs_ref):
    s = ss_ref[0, 0] + ss_ref[1, 0]
    r = lax.broadcasted_iota(_i32, (D, D), 0)
    c = lax.broadcasted_iota(_i32, (D, D), 1)
    eye = jnp.where(r == c, 1.0, 0.0).astype(_f32)
    srep = jnp.broadcast_to(s, (D, D))
    colb = lax.dot_general(eye, srep, (((1,), (1,)), ((), ())),
                           preferred_element_type=_f32)
    xs_ref[...] = xl_ref[...] / (colb + 1e-16)


_tc_div = pl.pallas_call(
    _tc_div_body,
    grid=(NP // D,),
    in_specs=[pl.BlockSpec((D, D), lambda i: (i, 0)),
              pl.BlockSpec((2, 1, 1, D), lambda i: (0, i, 0, 0))],
    out_specs=pl.BlockSpec((D, D), lambda i: (i, 0)),
    out_shape=jax.ShapeDtypeStruct((NP, D), _f32),
)


_mesh = plsc.VectorSubcoreMesh(core_axis_name="c", subcore_axis_name="s",
                               num_cores=NC, num_subcores=NS)


def _shuf(v, idx16):
    return jnp.take_along_axis(v, idx16, axis=0, mode="promise_in_bounds")


def _splat(v, l):
    return _shuf(v, jnp.full((16,), l, _i32))


def _sc_a_body(nl_hbm, nr_hbm, src_hbm, dst_hbm, w_hbm, ssum_hbm,
               si0, di0, si1, di1, srow0, srow1, rl0, rr0, rl1, rr1, wv,
               msg0, msg1, ssum_sp, semg0, semg1, semi, sems0, sems1):
    cid = lax.axis_index("c")
    sid = lax.axis_index("s")
    base = jnp.where(cid == 0, sid * CA, NS * CA + sid * CB)
    half = jnp.where(cid == 0, CA // 2, CB // 2)
    iota = lax.iota(_i32, 16)

    pltpu.sync_copy(src_hbm.at[base], si0)
    pltpu.sync_copy(dst_hbm.at[base], di0)
    pltpu.async_copy(nl_hbm.at[si0], rl0, semg0)
    pltpu.async_copy(nr_hbm.at[di0], rr0, semg0)
    z16 = jnp.zeros((16,), _f32)
    zi16 = jnp.zeros((16,), _i32)
    for e in range(CH):
        msg0[e, :] = z16
        msg1[e, :] = z16
    for g in range(CH // 16):
        srow0[pl.ds(g * 16, 16)] = zi16
        srow1[pl.ds(g * 16, 16)] = zi16
    pltpu.sync_copy(msg0.at[pl.ds(0, SROWS // NS)],
                    ssum_sp.at[pl.ds(sid * (SROWS // NS), SROWS // NS)])
    plsc.subcore_barrier()
    pltpu.async_copy(msg0, ssum_sp.at[srow0], add=True, sem=sems0)
    pltpu.async_copy(msg1, ssum_sp.at[srow1], add=True, sem=sems1)

    def _prefetch(j, sir, dir_):
        pltpu.async_copy(src_hbm.at[base + j], sir, semi)
        pltpu.async_copy(dst_hbm.at[base + j], dir_, semi)

    def _drain_prefetch(sir, dir_):
        pltpu.make_async_copy(src_hbm.at[0], sir, semi).wait()
        pltpu.make_async_copy(src_hbm.at[0], dir_, semi).wait()

    def compute(j, rl, rr, sir, msg, srow):
        jm = jnp.bitwise_and(j, 7)

        def group(g, carry):
            s16 = sir[pl.ds(g * 16, 16)]
            w_acc = jnp.zeros((16,), _f32)
            for l in range(16):
                e = g * 16 + l
                acc = rl[e, pl.ds(0, 16)] * rr[e, pl.ds(0, 16)]
                for q in range(1, D // 16):
                    acc = acc + (rl[e, pl.ds(q * 16, 16)]
                                 * rr[e, pl.ds(q * 16, 16)])
                for k in (8, 4, 2, 1):
                    acc = acc + _shuf(acc, jnp.bitwise_xor(iota, k))
                w_acc = jnp.where(iota == l, acc, w_acc)
            w16 = jnp.exp(jnp.exp(w_acc) * 4.0)
            wv[jm, pl.ds(g * 16, 16)] = w16
            srow[pl.ds(g * 16, 16)] = lax.shift_right_logical(s16, 4)
            slane = jnp.bitwise_and(s16, 15)
            for l in range(16):
                msg[g * 16 + l, :] = jnp.where(
                    iota == _splat(slane, l), _splat(w16, l), 0.0)
            return carry

        lax.fori_loop(0, CH // 16, group, 0)

        @pl.when(jm == 7)
        def _():
            off = pl.multiple_of(base + j - 7, 8)
            pltpu.sync_copy(wv, w_hbm.at[pl.ds(off, 8)])

    def body(jj, carry):
        a = 2 * jj
        _prefetch(a + 1, si1, di1)
        pltpu.make_async_copy(nl_hbm.at[pl.ds(0, CH)], rl0, semg0).wait()
        pltpu.make_async_copy(nl_hbm.at[pl.ds(0, CH)], rr0, semg0).wait()
        _drain_prefetch(si1, di1)
        pltpu.async_copy(nl_hbm.at[si1], rl1, semg1)
        pltpu.async_copy(nr_hbm.at[di1], rr1, semg1)
        pltpu.make_async_copy(msg0, ssum_sp.at[srow0], sems0).wait()
        compute(a, rl0, rr0, si0, msg0, srow0)
        pltpu.async_copy(msg0, ssum_sp.at[srow0], add=True, sem=sems0)
        pltpu.make_async_copy(nl_hbm.at[pl.ds(0, CH)], rl1, semg1).wait()
        pltpu.make_async_copy(nl_hbm.at[pl.ds(0, CH)], rr1, semg1).wait()

        @pl.when(jj < half - 1)
        def _():
            _prefetch(a + 2, si0, di0)
            _drain_prefetch(si0, di0)
            pltpu.async_copy(nl_hbm.at[si0], rl0, semg0)
            pltpu.async_copy(nr_hbm.at[di0], rr0, semg0)

        pltpu.make_async_copy(msg1, ssum_sp.at[srow1], sems1).wait()
        compute(a + 1, rl1, rr1, si1, msg1, srow1)
        pltpu.async_copy(msg1, ssum_sp.at[srow1], add=True, sem=sems1)
        return carry

    lax.fori_loop(0, half, body, 0)

    pltpu.make_async_copy(msg0, ssum_sp.at[srow0], sems0).wait()
    pltpu.make_async_copy(msg1, ssum_sp.at[srow1], sems1).wait()
    plsc.subcore_barrier()
    per = SROWS // NS
    pltpu.sync_copy(ssum_sp.at[pl.ds(sid * per, per)],
                    ssum_hbm.at[cid].at[pl.ds(sid * per, per)])


_sc_pass_a = functools.partial(
    pl.kernel,
    out_type=[jax.ShapeDtypeStruct((TOTC, CH), _f32),
              jax.ShapeDtypeStruct((NC, SROWS, 16), _f32)],
    mesh=_mesh,
    scratch_types=[
        pltpu.VMEM((CH,), _i32),
        pltpu.VMEM((CH,), _i32),
        pltpu.VMEM((CH,), _i32),
        pltpu.VMEM((CH,), _i32),
        pltpu.VMEM((CH,), _i32),
        pltpu.VMEM((CH,), _i32),
        pltpu.VMEM((CH, D), _f32),
        pltpu.VMEM((CH, D), _f32),
        pltpu.VMEM((CH, D), _f32),
        pltpu.VMEM((CH, D), _f32),
        pltpu.VMEM((8, CH), _f32),
        pltpu.VMEM((CH, 16), _f32),
        pltpu.VMEM((CH, 16), _f32),
        pltpu.VMEM_SHARED((SROWS, 16), _f32),
        pltpu.SemaphoreType.DMA,
        pltpu.SemaphoreType.DMA,
        pltpu.SemaphoreType.DMA,
        pltpu.SemaphoreType.DMA,
        pltpu.SemaphoreType.DMA,
    ],
)(_sc_a_body)


def _sc_b_body(xs_hbm, w_hbm, src_hbm, dst_hbm, out_hbm,
               si0, di0, wv0, si1, di1, wv1, rows0, rows1, out_sp,
               semg0, semg1, semi, sems0, sems1):
    cid = lax.axis_index("c")
    sid = lax.axis_index("s")
    base = jnp.where(cid == 0, sid * CA, NS * CA + sid * CB)
    half = jnp.where(cid == 0, CA // 2, CB // 2)
    z16 = jnp.zeros((16,), _f32)

    pltpu.sync_copy(src_hbm.at[base], si0)
    pltpu.sync_copy(dst_hbm.at[base], di0)
    pltpu.sync_copy(w_hbm.at[base], wv0)
    pltpu.async_copy(xs_hbm.at[si0], rows0, semg0)
    for r in range(CH):
        for q in range(D // 16):
            rows1[r, pl.ds(q * 16, 16)] = z16
    for t in range(NP // NS // CH):
        pltpu.sync_copy(rows1,
                        out_sp.at[pl.ds(sid * (NP // NS) + t * CH, CH)])
    plsc.subcore_barrier()
    pltpu.async_copy(rows1, out_sp.at[di0], add=True, sem=sems1)

    def scale(rows, wvr):
        for g in range(CH // 16):
            wv16 = wvr[pl.ds(g * 16, 16)]
            for l in range(16):
                e = g * 16 + l
                wts = _splat(wv16, l)
                for q in range(D // 16):
                    rows[e, pl.ds(q * 16, 16)] = (
                        rows[e, pl.ds(q * 16, 16)] * wts)

    def _prefetch(j, sir, dir_, wvr):
        pltpu.async_copy(src_hbm.at[base + j], sir, semi)
        pltpu.async_copy(dst_hbm.at[base + j], dir_, semi)
        pltpu.async_copy(w_hbm.at[base + j], wvr, semi)

    def _drain_prefetch(sir, dir_, wvr):
        pltpu.make_async_copy(src_hbm.at[0], sir, semi).wait()
        pltpu.make_async_copy(src_hbm.at[0], dir_, semi).wait()
        pltpu.make_async_copy(w_hbm.at[0], wvr, semi).wait()

    def body(jj, carry):
        a = 2 * jj
        pltpu.make_async_copy(rows1, out_sp.at[di0], sems1).wait()
        _prefetch(a + 1, si1, di1, wv1)
        pltpu.make_async_copy(xs_hbm.at[pl.ds(0, CH)], rows0, semg0).wait()
        _drain_prefetch(si1, di1, wv1)
        pltpu.async_copy(xs_hbm.at[si1], rows1, semg1)
        scale(rows0, wv0)
        pltpu.async_copy(rows0, out_sp.at[di0], add=True, sem=sems0)
        pltpu.make_async_copy(xs_hbm.at[pl.ds(0, CH)], rows1, semg1).wait()

        @pl.when(jj < half - 1)
        def _():
            pltpu.make_async_copy(rows0, out_sp.at[di0], sems0).wait()
            _prefetch(a + 2, si0, di0, wv0)
            _drain_prefetch(si0, di0, wv0)
            pltpu.async_copy(xs_hbm.at[si0], rows0, semg0)

        scale(rows1, wv1)
        pltpu.async_copy(rows1, out_sp.at[di1], add=True, sem=sems1)
        return carry

    lax.fori_loop(0, half, body, 0)

    pltpu.make_async_copy(rows0, out_sp.at[di0], sems0).wait()
    pltpu.make_async_copy(rows1, out_sp.at[di1], sems1).wait()
    plsc.subcore_barrier()
    for t in range(NP // NS // 64):
        off = sid * (NP // NS) + t * 64
        pltpu.sync_copy(out_sp.at[pl.ds(off, 64)],
                        out_hbm.at[cid].at[pl.ds(off, 64)])


_sc_pass_b = functools.partial(
    pl.kernel,
    out_type=jax.ShapeDtypeStruct((NC, NP, D), _f32),
    mesh=_mesh,
    scratch_types=[
        pltpu.VMEM((CH,), _i32),
        pltpu.VMEM((CH,), _i32),
        pltpu.VMEM((CH,), _f32),
        pltpu.VMEM((CH,), _i32),
        pltpu.VMEM((CH,), _i32),
        pltpu.VMEM((CH,), _f32),
        pltpu.VMEM((CH, D), _f32),
        pltpu.VMEM((CH, D), _f32),
        pltpu.VMEM_SHARED((NP, D), _f32),
        pltpu.SemaphoreType.DMA,
        pltpu.SemaphoreType.DMA,
        pltpu.SemaphoreType.DMA,
        pltpu.SemaphoreType.DMA,
        pltpu.SemaphoreType.DMA,
    ],
)(_sc_b_body)


def kernel(x, edge_index, W_in, b_in, Wl1, Wr1, bias1, Wc1, Wl2, Wr2, bias2,
           Wc2, Wm1, bm1, Wm2, bm2):
    x_pad = jnp.pad(x, ((0, NP - N), (0, 0)))
    pad = jnp.full((E_PAD - E,), N, _i32)
    srcE = jnp.concatenate([edge_index[0].astype(_i32), pad]).reshape(
        TOTC, CH)
    dstE = jnp.concatenate([edge_index[1].astype(_i32), pad]).reshape(
        TOTC, CH)

    nl1, nr1, xl1 = _tc_pre(x_pad, W_in, b_in.reshape(1, D), Wl1, Wr1)
    w1, ssum1 = _sc_pass_a(nl1, nr1, srcE, dstE)
    xs1 = _tc_div(xl1, ssum1.reshape(NC, NP // D, 1, D))
    outP1 = _sc_pass_b(xs1, w1, srcE, dstE)

    nl2, nr2, xl2 = _tc_mid(outP1, bias1.reshape(1, D), Wc1, Wl2, Wr2)
    w2, ssum2 = _sc_pass_a(nl2, nr2, srcE, dstE)
    xs2 = _tc_div(xl2, ssum2.reshape(NC, NP // D, 1, D))
    outP2 = _sc_pass_b(xs2, w2, srcE, dstE)

    z = _tc_fin(outP2, bias2.reshape(1, D), Wc2, Wm1, bm1.reshape(1, D),
                Wm2, bm2.reshape(1, D))
    return z[:N]


# speedup vs baseline: 1.0010x; 1.0010x over previous
import functools

import jax
import jax.numpy as jnp
from jax import lax
from jax.experimental import pallas as pl
from jax.experimental.pallas import tpu as pltpu
from jax.experimental.pallas import tpu_sc as plsc

N = 10000
D = 128
E = 320000
NC, NS = 2, 16
NW = NC * NS
CH = 128
C = -(-E // (NW * CH))
C += C % 2
EW = C * CH
E_PAD = NW * EW
NP = 10240
SROWS = NP // 16
TOTC = NW * C
CA = C
CB = TOTC // NS - CA

_i32 = jnp.int32
_f32 = jnp.float32


def _leaky(v):
    return jnp.where(v >= 0, v, 0.01 * v)


def _norm_rows(v):
    n = jnp.sqrt(jnp.sum(v * v, axis=-1, keepdims=True))
    return v / jnp.maximum(n, 1e-12)


def _tc_pre_body(x_ref, win_ref, bin_ref, wl_ref, wr_ref, nl_ref, nr_ref,
                 xl_ref):
    h = _leaky(jnp.dot(x_ref[...], win_ref[...],
                       preferred_element_type=_f32) + bin_ref[...])
    xl = jnp.dot(h, wl_ref[...], preferred_element_type=_f32)
    xr = jnp.dot(h, wr_ref[...], preferred_element_type=_f32)
    xl_ref[...] = xl
    nl_ref[...] = _norm_rows(xl)
    nr_ref[...] = _norm_rows(xr)


def _tc_mid_body(op_ref, b_ref, wc_ref, wl_ref, wr_ref, nl_ref, nr_ref,
                 xl_ref):
    o = op_ref[0] + op_ref[1] + b_ref[...]
    h = _leaky(jnp.dot(o, wc_ref[...], preferred_element_type=_f32))
    xl = jnp.dot(h, wl_ref[...], preferred_element_type=_f32)
    xr = jnp.dot(h, wr_ref[...], preferred_element_type=_f32)
    xl_ref[...] = xl
    nl_ref[...] = _norm_rows(xl)
    nr_ref[...] = _norm_rows(xr)


def _tc_fin_body(op_ref, b_ref, wc_ref, wm1_ref, bm1_ref, wm2_ref, bm2_ref,
                 z_ref):
    o = op_ref[0] + op_ref[1] + b_ref[...]
    h = _leaky(jnp.dot(o, wc_ref[...], preferred_element_type=_f32))
    m = jnp.maximum(jnp.dot(h, wm1_ref[...], preferred_element_type=_f32)
                    + bm1_ref[...], 0.0)
    z_ref[...] = (jnp.dot(m, wm2_ref[...], preferred_element_type=_f32)
                  + bm2_ref[...])


_GRID = 16
_RB = NP // _GRID

_row_spec = pl.BlockSpec((_RB, D), lambda i: (i, 0))
_w_spec = pl.BlockSpec((D, D), lambda i: (0, 0))
_b_spec = pl.BlockSpec((1, D), lambda i: (0, 0))
_op_spec = pl.BlockSpec((2, _RB, D), lambda i: (0, i, 0))

_tc_pre = pl.pallas_call(
    _tc_pre_body,
    grid=(_GRID,),
    in_specs=[_row_spec, _w_spec, _b_spec, _w_spec, _w_spec],
    out_specs=[_row_spec, _row_spec, _row_spec],
    out_shape=[jax.ShapeDtypeStruct((NP, D), _f32)] * 3,
)

_oph_spec = pl.BlockSpec((2, _RB, D), lambda i: (0, i, 0))

_tc_mid = pl.pallas_call(
    _tc_mid_body,
    grid=(_GRID,),
    in_specs=[_oph_spec, _b_spec, _w_spec, _w_spec, _w_spec],
    out_specs=[_row_spec, _row_spec, _row_spec],
    out_shape=[jax.ShapeDtypeStruct((NP, D), _f32)] * 3,
)

_tc_fin = pl.pallas_call(
    _tc_fin_body,
    grid=(_GRID,),
    in_specs=[_oph_spec, _b_spec, _w_spec, _w_spec, _b_spec, _w_spec,
              _b_spec],
    out_specs=_row_spec,
    out_shape=jax.ShapeDtypeStruct((NP, D), _f32),
)


def _tc_div_body(xl_ref, ss_ref, xs_ref):
    s = ss_ref[0, 0] + ss_ref[1, 0]
    r = lax.broadcasted_iota(_i32, (D, D), 0)
    c = lax.broadcasted_iota(_i32, (D, D), 1)
    eye = jnp.where(r == c, 1.0, 0.0).astype(_f32)
    srep = jnp.broadcast_to(s, (D, D))
    colb = lax.dot_general(eye, srep, (((1,), (1,)), ((), ())),
                           preferred_element_type=_f32)
    xs_ref[...] = xl_ref[...] / (colb + 1e-16)


_tc_div = pl.pallas_call(
    _tc_div_body,
    grid=(NP // D,),
    in_specs=[pl.BlockSpec((D, D), lambda i: (i, 0)),
              pl.BlockSpec((2, 1, 1, D), lambda i: (0, i, 0, 0))],
    out_specs=pl.BlockSpec((D, D), lambda i: (i, 0)),
    out_shape=jax.ShapeDtypeStruct((NP, D), _f32),
)


_mesh = plsc.VectorSubcoreMesh(core_axis_name="c", subcore_axis_name="s",
                               num_cores=NC, num_subcores=NS)


def _shuf(v, idx16):
    return jnp.take_along_axis(v, idx16, axis=0, mode="promise_in_bounds")


def _splat(v, l):
    return _shuf(v, jnp.full((16,), l, _i32))


def _sc_a_body(nl_hbm, nr_hbm, src_hbm, dst_hbm, w_hbm, ssum_hbm,
               si0, di0, si1, di1, srow0, srow1, rl0, rr0, rl1, rr1, wv,
               msg0, msg1, ssum_sp, semg0, semg1, semi, sems0, sems1):
    cid = lax.axis_index("c")
    sid = lax.axis_index("s")
    base = jnp.where(cid == 0, sid * CA, NS * CA + sid * CB)
    half = jnp.where(cid == 0, CA // 2, CB // 2)
    iota = lax.iota(_i32, 16)

    pltpu.sync_copy(src_hbm.at[base], si0)
    pltpu.sync_copy(dst_hbm.at[base], di0)
    pltpu.async_copy(nl_hbm.at[si0], rl0, semg0)
    pltpu.async_copy(nr_hbm.at[di0], rr0, semg0)
    z16 = jnp.zeros((16,), _f32)
    zi16 = jnp.zeros((16,), _i32)
    for e in range(CH):
        msg0[e, :] = z16
        msg1[e, :] = z16
    for g in range(CH // 16):
        srow0[pl.ds(g * 16, 16)] = zi16
        srow1[pl.ds(g * 16, 16)] = zi16
    pltpu.sync_copy(msg0.at[pl.ds(0, SROWS // NS)],
                    ssum_sp.at[pl.ds(sid * (SROWS // NS), SROWS // NS)])
    plsc.subcore_barrier()
    pltpu.async_copy(msg0, ssum_sp.at[srow0], add=True, sem=sems0)
    pltpu.async_copy(msg1, ssum_sp.at[srow1], add=True, sem=sems1)

    def _prefetch(j, sir, dir_):
        pltpu.async_copy(src_hbm.at[base + j], sir, semi)
        pltpu.async_copy(dst_hbm.at[base + j], dir_, semi)

    def _drain_prefetch(sir, dir_):
        pltpu.make_async_copy(src_hbm.at[0], sir, semi).wait()
        pltpu.make_async_copy(src_hbm.at[0], dir_, semi).wait()

    def compute(j, rl, rr, sir, msg, srow):
        jm = jnp.bitwise_and(j, 7)

        def group(g, carry):
            s16 = sir[pl.ds(g * 16, 16)]
            w_acc = jnp.zeros((16,), _f32)
            for l in range(16):
                e = g * 16 + l
                acc = rl[e, pl.ds(0, 16)] * rr[e, pl.ds(0, 16)]
                for q in range(1, D // 16):
                    acc = acc + (rl[e, pl.ds(q * 16, 16)]
                                 * rr[e, pl.ds(q * 16, 16)])
                for k in (8, 4, 2, 1):
                    acc = acc + _shuf(acc, jnp.bitwise_xor(iota, k))
                w_acc = jnp.where(iota == l, acc, w_acc)
            w16 = jnp.exp(jnp.exp(w_acc) * 4.0)
            wv[jm, pl.ds(g * 16, 16)] = w16
            srow[pl.ds(g * 16, 16)] = lax.shift_right_logical(s16, 4)
            slane = jnp.bitwise_and(s16, 15)
            for l in range(16):
                msg[g * 16 + l, :] = jnp.where(
                    iota == _splat(slane, l), _splat(w16, l), 0.0)
            return carry

        lax.fori_loop(0, CH // 16, group, 0)

        @pl.when(jm == 7)
        def _():
            off = pl.multiple_of(base + j - 7, 8)
            pltpu.sync_copy(wv, w_hbm.at[pl.ds(off, 8)])

    def body(jj, carry):
        a = 2 * jj
        _prefetch(a + 1, si1, di1)
        pltpu.make_async_copy(nl_hbm.at[pl.ds(0, CH)], rl0, semg0).wait()
        pltpu.make_async_copy(nl_hbm.at[pl.ds(0, CH)], rr0, semg0).wait()
        _drain_prefetch(si1, di1)
        pltpu.async_copy(nl_hbm.at[si1], rl1, semg1)
        pltpu.async_copy(nr_hbm.at[di1], rr1, semg1)
        pltpu.make_async_copy(msg0, ssum_sp.at[srow0], sems0).wait()
        compute(a, rl0, rr0, si0, msg0, srow0)
        pltpu.async_copy(msg0, ssum_sp.at[srow0], add=True, sem=sems0)
        pltpu.make_async_copy(nl_hbm.at[pl.ds(0, CH)], rl1, semg1).wait()
        pltpu.make_async_copy(nl_hbm.at[pl.ds(0, CH)], rr1, semg1).wait()

        @pl.when(jj < half - 1)
        def _():
            _prefetch(a + 2, si0, di0)
            _drain_prefetch(si0, di0)
            pltpu.async_copy(nl_hbm.at[si0], rl0, semg0)
            pltpu.async_copy(nr_hbm.at[di0], rr0, semg0)

        pltpu.make_async_copy(msg1, ssum_sp.at[srow1], sems1).wait()
        compute(a + 1, rl1, rr1, si1, msg1, srow1)
        pltpu.async_copy(msg1, ssum_sp.at[srow1], add=True, sem=sems1)
        return carry

    lax.fori_loop(0, half, body, 0)

    pltpu.make_async_copy(msg0, ssum_sp.at[srow0], sems0).wait()
    pltpu.make_async_copy(msg1, ssum_sp.at[srow1], sems1).wait()
    plsc.subcore_barrier()
    per = SROWS // NS
    pltpu.sync_copy(ssum_sp.at[pl.ds(sid * per, per)],
                    ssum_hbm.at[cid].at[pl.ds(sid * per, per)])


_sc_pass_a = functools.partial(
    pl.kernel,
    out_type=[jax.ShapeDtypeStruct((TOTC, CH), _f32),
              jax.ShapeDtypeStruct((NC, SROWS, 16), _f32)],
    mesh=_mesh,
    scratch_types=[
        pltpu.VMEM((CH,), _i32),
        pltpu.VMEM((CH,), _i32),
        pltpu.VMEM((CH,), _i32),
        pltpu.VMEM((CH,), _i32),
        pltpu.VMEM((CH,), _i32),
        pltpu.VMEM((CH,), _i32),
        pltpu.VMEM((CH, D), _f32),
        pltpu.VMEM((CH, D), _f32),
        pltpu.VMEM((CH, D), _f32),
        pltpu.VMEM((CH, D), _f32),
        pltpu.VMEM((8, CH), _f32),
        pltpu.VMEM((CH, 16), _f32),
        pltpu.VMEM((CH, 16), _f32),
        pltpu.VMEM_SHARED((SROWS, 16), _f32),
        pltpu.SemaphoreType.DMA,
        pltpu.SemaphoreType.DMA,
        pltpu.SemaphoreType.DMA,
        pltpu.SemaphoreType.DMA,
        pltpu.SemaphoreType.DMA,
    ],
)(_sc_a_body)


def _sc_b_body(xs_hbm, w_hbm, src_hbm, dst_hbm, out_hbm,
               si0, di0, wv0, si1, di1, wv1, rows0, rows1, zb, out_sp,
               semg0, semg1, semi):
    cid = lax.axis_index("c")
    sid = lax.axis_index("s")
    base = jnp.where(cid == 0, sid * CA, NS * CA + sid * CB)
    half = jnp.where(cid == 0, CA // 2, CB // 2)
    z16 = jnp.zeros((16,), _f32)

    for r in range(64):
        for q in range(D // 16):
            zb[r, pl.ds(q * 16, 16)] = z16
    for t in range(NP // NS // 64):
        pltpu.sync_copy(zb, out_sp.at[pl.ds(sid * (NP // NS) + t * 64, 64)])
    pltpu.sync_copy(src_hbm.at[base], si0)
    pltpu.sync_copy(dst_hbm.at[base], di0)
    pltpu.sync_copy(w_hbm.at[base], wv0)
    pltpu.async_copy(xs_hbm.at[si0], rows0, semg0)
    plsc.subcore_barrier()

    def scale(rows, wvr):
        for g in range(CH // 16):
            wv16 = wvr[pl.ds(g * 16, 16)]
            for l in range(16):
                e = g * 16 + l
                wts = _splat(wv16, l)
                for q in range(D // 16):
                    rows[e, pl.ds(q * 16, 16)] = (
                        rows[e, pl.ds(q * 16, 16)] * wts)

    def _prefetch(j, sir, dir_, wvr):
        pltpu.async_copy(src_hbm.at[base + j], sir, semi)
        pltpu.async_copy(dst_hbm.at[base + j], dir_, semi)
        pltpu.async_copy(w_hbm.at[base + j], wvr, semi)

    def _drain_prefetch(sir, dir_, wvr):
        pltpu.make_async_copy(src_hbm.at[0], sir, semi).wait()
        pltpu.make_async_copy(src_hbm.at[0], dir_, semi).wait()
        pltpu.make_async_copy(w_hbm.at[0], wvr, semi).wait()

    def body(jj, carry):
        a = 2 * jj
        _prefetch(a + 1, si1, di1, wv1)
        pltpu.make_async_copy(xs_hbm.at[pl.ds(0, CH)], rows0, semg0).wait()
        _drain_prefetch(si1, di1, wv1)
        pltpu.async_copy(xs_hbm.at[si1], rows1, semg1)
        scale(rows0, wv0)
        pltpu.sync_copy(rows0, out_sp.at[di0], add=True)
        pltpu.make_async_copy(xs_hbm.at[pl.ds(0, CH)], rows1, semg1).wait()

        @pl.when(jj < half - 1)
        def _():
            _prefetch(a + 2, si0, di0, wv0)
            _drain_prefetch(si0, di0, wv0)
            pltpu.async_copy(xs_hbm.at[si0], rows0, semg0)

        scale(rows1, wv1)
        pltpu.sync_copy(rows1, out_sp.at[di1], add=True)
        return carry

    lax.fori_loop(0, half, body, 0)

    plsc.subcore_barrier()
    for t in range(NP // NS // 64):
        off = sid * (NP // NS) + t * 64
        pltpu.sync_copy(out_sp.at[pl.ds(off, 64)],
                        out_hbm.at[cid].at[pl.ds(off, 64)])


_sc_pass_b = functools.partial(
    pl.kernel,
    out_type=jax.ShapeDtypeStruct((NC, NP, D), _f32),
    mesh=_mesh,
    scratch_types=[
        pltpu.VMEM((CH,), _i32),
        pltpu.VMEM((CH,), _i32),
        pltpu.VMEM((CH,), _f32),
        pltpu.VMEM((CH,), _i32),
        pltpu.VMEM((CH,), _i32),
        pltpu.VMEM((CH,), _f32),
        pltpu.VMEM((CH, D), _f32),
        pltpu.VMEM((CH, D), _f32),
        pltpu.VMEM((64, D), _f32),
        pltpu.VMEM_SHARED((NP, D), _f32),
        pltpu.SemaphoreType.DMA,
        pltpu.SemaphoreType.DMA,
        pltpu.SemaphoreType.DMA,
    ],
)(_sc_b_body)


def kernel(x, edge_index, W_in, b_in, Wl1, Wr1, bias1, Wc1, Wl2, Wr2, bias2,
           Wc2, Wm1, bm1, Wm2, bm2):
    x_pad = jnp.pad(x, ((0, NP - N), (0, 0)))
    pad = jnp.full((E_PAD - E,), N, _i32)
    srcE = jnp.concatenate([edge_index[0].astype(_i32), pad]).reshape(
        TOTC, CH)
    dstE = jnp.concatenate([edge_index[1].astype(_i32), pad]).reshape(
        TOTC, CH)

    nl1, nr1, xl1 = _tc_pre(x_pad, W_in, b_in.reshape(1, D), Wl1, Wr1)
    w1, ssum1 = _sc_pass_a(nl1, nr1, srcE, dstE)
    xs1 = _tc_div(xl1, ssum1.reshape(NC, NP // D, 1, D))
    outP1 = _sc_pass_b(xs1, w1, srcE, dstE)

    nl2, nr2, xl2 = _tc_mid(outP1, bias1.reshape(1, D), Wc1, Wl2, Wr2)
    w2, ssum2 = _sc_pass_a(nl2, nr2, srcE, dstE)
    xs2 = _tc_div(xl2, ssum2.reshape(NC, NP // D, 1, D))
    outP2 = _sc_pass_b(xs2, w2, srcE, dstE)

    z = _tc_fin(outP2, bias2.reshape(1, D), Wc2, Wm1, bm1.reshape(1, D),
                Wm2, bm2.reshape(1, D))
    return z[:N]
